# baseline (device time: 70062 ns/iter reference)
import jax
import jax.numpy as jnp
from jax import lax
from jax.experimental import pallas as pl
from jax.experimental.pallas import tpu as pltpu

N_DEV = 4


def kernel(dy, W):
    m, k = dy.shape
    n = W.shape[0]
    chunk = m // N_DEV

    def body(dy_ref, w_ref, out_ref, stage_ref, rs_recv, ag_recv,
             rs_send_sems, rs_recv_sems, ag_send_sems, ag_recv_sems):
        my = lax.axis_index("i")
        left = (my - 1) % N_DEV
        right = (my + 1) % N_DEV

        barrier_sem = pltpu.get_barrier_semaphore()
        for nbr in (left, right):
            pl.semaphore_signal(
                barrier_sem, inc=1,
                device_id=(nbr,), device_id_type=pl.DeviceIdType.MESH,
            )
        pl.semaphore_wait(barrier_sem, 2)

        partial = lax.dot_general(
            dy_ref[...].astype(jnp.bfloat16),
            w_ref[...].astype(jnp.bfloat16),
            dimension_numbers=(((1,), (1,)), ((), ())),
            preferred_element_type=jnp.float32,
        )
        out_ref[...] = partial

        def load_chunk(idx):
            return out_ref[pl.ds(idx * chunk, chunk), :]

        own_idx = (my + 1) % N_DEV
        stage_ref[...] = load_chunk(my % N_DEV).astype(jnp.bfloat16)
        for h in range(N_DEV - 1):
            rdma = pltpu.make_async_remote_copy(
                src_ref=stage_ref,
                dst_ref=rs_recv.at[h],
                send_sem=rs_send_sems.at[h],
                recv_sem=rs_recv_sems.at[h],
                device_id=(right,),
                device_id_type=pl.DeviceIdType.MESH,
            )
            rdma.start()
            rdma.wait()
            recv_idx = (my - h - 1) % N_DEV
            red = rs_recv[h].astype(jnp.float32) + load_chunk(recv_idx)
            stage_ref[...] = red.astype(jnp.bfloat16)
        out_ref[pl.ds(own_idx * chunk, chunk), :] = stage_ref[...].astype(
            jnp.float32)

        for h in range(N_DEV - 1):
            src = stage_ref if h == 0 else ag_recv.at[h - 1]
            rdma = pltpu.make_async_remote_copy(
                src_ref=src,
                dst_ref=ag_recv.at[h],
                send_sem=ag_send_sems.at[h],
                recv_sem=ag_recv_sems.at[h],
                device_id=(right,),
                device_id_type=pl.DeviceIdType.MESH,
            )
            rdma.start()
            rdma.wait()
            got_idx = (my - h) % N_DEV
            out_ref[pl.ds(got_idx * chunk, chunk), :] = ag_recv[h].astype(
                jnp.float32)

    return pl.pallas_call(
        body,
        out_shape=jax.ShapeDtypeStruct((m, n), jnp.float32),
        in_specs=[
            pl.BlockSpec(memory_space=pltpu.VMEM),
            pl.BlockSpec(memory_space=pltpu.VMEM),
        ],
        out_specs=pl.BlockSpec(memory_space=pltpu.VMEM),
        scratch_shapes=[
            pltpu.VMEM((chunk, n), jnp.bfloat16),
            pltpu.VMEM((N_DEV - 1, chunk, n), jnp.bfloat16),
            pltpu.VMEM((N_DEV - 1, chunk, n), jnp.bfloat16),
            pltpu.SemaphoreType.DMA((N_DEV - 1,)),
            pltpu.SemaphoreType.DMA((N_DEV - 1,)),
            pltpu.SemaphoreType.DMA((N_DEV - 1,)),
            pltpu.SemaphoreType.DMA((N_DEV - 1,)),
        ],
        compiler_params=pltpu.CompilerParams(collective_id=0),
    )(dy, W)


# device time: 23031 ns/iter; 3.0421x vs baseline; 3.0421x over previous
import jax
import jax.numpy as jnp
from jax import lax
from jax.experimental import pallas as pl
from jax.experimental.pallas import tpu as pltpu

N_DEV = 4


def kernel(dy, W):
    m, _ = dy.shape
    n = W.shape[0]
    half = m // 4
    quar = m // 8

    def body(dy_ref, w_ref, out_ref, acc, r1a, r1b, r2a, r2b, ssems, rsems):
        my = lax.axis_index("i")
        bit0 = my & 1
        bit1 = (my >> 1) & 1
        p1 = my ^ 1
        p3 = my ^ 3

        ka = bit0 ^ bit1
        ma = bit0
        kb = bit1
        mb = bit0

        barrier_sem = pltpu.get_barrier_semaphore()
        for nbr in (p1, p3):
            pl.semaphore_signal(
                barrier_sem, inc=1,
                device_id=(nbr,), device_id_type=pl.DeviceIdType.MESH,
            )
        pl.semaphore_wait(barrier_sem, 2)

        acc[...] = lax.dot_general(
            dy_ref[...].astype(jnp.bfloat16),
            w_ref[...].astype(jnp.bfloat16),
            dimension_numbers=(((1,), (1,)), ((), ())),
            preferred_element_type=jnp.float32,
        ).astype(jnp.bfloat16)

        a_my_half = ka * half
        a_send_half = (1 - ka) * half
        a_my_q = ka * half + ma * quar
        a_send_q = ka * half + (1 - ma) * quar
        b0 = 2 * half
        b_my_half = b0 + kb * half
        b_send_half = b0 + (1 - kb) * half
        b_my_q = b0 + kb * half + mb * quar
        b_send_q = b0 + kb * half + (1 - mb) * quar

        def xchg(src, dst, idx, tgt):
            r = pltpu.make_async_remote_copy(
                src_ref=src, dst_ref=dst,
                send_sem=ssems.at[idx], recv_sem=rsems.at[idx],
                device_id=(tgt,), device_id_type=pl.DeviceIdType.MESH,
            )
            r.start()
            return r

        def reduce_rows(off, nrows, rbuf):
            cur = acc[pl.ds(off, nrows), :].astype(jnp.float32)
            acc[pl.ds(off, nrows), :] = (
                cur + rbuf[...].astype(jnp.float32)
            ).astype(jnp.bfloat16)

        ra = xchg(acc.at[pl.ds(a_send_half, half)], r1a, 0, p1)
        rb = xchg(acc.at[pl.ds(b_send_half, half)], r1b, 1, p3)
        ra.wait()
        rb.wait()
        reduce_rows(a_my_half, half, r1a)
        reduce_rows(b_my_half, half, r1b)

        ra = xchg(acc.at[pl.ds(a_send_q, quar)], r2a, 2, p3)
        rb = xchg(acc.at[pl.ds(b_send_q, quar)], r2b, 3, p1)
        ra.wait()
        rb.wait()
        reduce_rows(a_my_q, quar, r2a)
        reduce_rows(b_my_q, quar, r2b)

        ra = xchg(acc.at[pl.ds(a_my_q, quar)], acc.at[pl.ds(a_my_q, quar)],
                  4, p3)
        rb = xchg(acc.at[pl.ds(b_my_q, quar)], acc.at[pl.ds(b_my_q, quar)],
                  5, p1)
        ra.wait()
        rb.wait()

        ra = xchg(acc.at[pl.ds(a_my_half, half)],
                  acc.at[pl.ds(a_my_half, half)], 6, p1)
        rb = xchg(acc.at[pl.ds(b_my_half, half)],
                  acc.at[pl.ds(b_my_half, half)], 7, p3)
        ra.wait()
        rb.wait()

        out_ref[...] = acc[...].astype(jnp.float32)

    return pl.pallas_call(
        body,
        out_shape=jax.ShapeDtypeStruct((m, n), jnp.float32),
        in_specs=[
            pl.BlockSpec(memory_space=pltpu.VMEM),
            pl.BlockSpec(memory_space=pltpu.VMEM),
        ],
        out_specs=pl.BlockSpec(memory_space=pltpu.VMEM),
        scratch_shapes=[
            pltpu.VMEM((m, n), jnp.bfloat16),
            pltpu.VMEM((half, n), jnp.bfloat16),
            pltpu.VMEM((half, n), jnp.bfloat16),
            pltpu.VMEM((quar, n), jnp.bfloat16),
            pltpu.VMEM((quar, n), jnp.bfloat16),
            pltpu.SemaphoreType.DMA((8,)),
            pltpu.SemaphoreType.DMA((8,)),
        ],
        compiler_params=pltpu.CompilerParams(collective_id=0),
    )(dy, W)
